# baseline (speedup 1.0000x reference)
import functools

import jax
import jax.numpy as jnp
from jax.experimental import pallas as pl

N, D, K = 16384, 128, 1000
K_PAD = 1024
TN = 256

_PAD_VAL = 1e15


def _kmeans_body(x_ref, c_ref, dist_ref, assign_ref):
    x = x_ref[...]
    c = c_ref[...]
    x2 = jnp.sum(x * x, axis=1, keepdims=True)
    c2 = jnp.sum(c * c, axis=1)[None, :]
    xc = jax.lax.dot_general(
        x, c, (((1,), (1,)), ((), ())),
        preferred_element_type=jnp.float32)
    sq = x2 + c2 - 2.0 * xc
    dist = jnp.sqrt(jnp.clip(sq, 1e-12))
    dist_ref[...] = dist[:, :K]
    assign_ref[...] = jnp.argmin(dist, axis=1).astype(jnp.int32)


@jax.jit
def kernel(data, centroids):
    c_pad = jnp.full((K_PAD, D), _PAD_VAL, jnp.float32).at[:K].set(centroids)
    grid = (N // TN,)
    dist, assign = pl.pallas_call(
        _kmeans_body,
        grid=grid,
        in_specs=[
            pl.BlockSpec((TN, D), lambda i: (i, 0)),
            pl.BlockSpec((K_PAD, D), lambda i: (0, 0)),
        ],
        out_specs=[
            pl.BlockSpec((TN, K), lambda i: (i, 0)),
            pl.BlockSpec((TN,), lambda i: (i,)),
        ],
        out_shape=[
            jax.ShapeDtypeStruct((N, K), jnp.float32),
            jax.ShapeDtypeStruct((N,), jnp.int32),
        ],
    )(data, c_pad)
    return dist, assign


# speedup vs baseline: 2.1426x; 2.1426x over previous
import jax
import jax.numpy as jnp
from jax.experimental import pallas as pl
from jax.experimental.pallas import tpu as pltpu

N, D, K = 16384, 128, 1000
K_PAD = 1024
TN = 256

_PAD_VAL = 1e15


def _kmeans_body(x_ref, cs_ref, dist_ref, assign_ref, c2_ref):
    @pl.when(pl.program_id(0) == 0)
    def _():
        cs0 = cs_ref[...]
        c2_ref[...] = 0.25 * jnp.sum(cs0 * cs0, axis=1, keepdims=True)

    x = x_ref[...]
    cs = cs_ref[...]
    xc2 = jax.lax.dot_general(
        cs, x, (((1,), (1,)), ((), ())),
        preferred_element_type=jnp.float32)
    x2 = jnp.sum(x * x, axis=1)
    sq = (c2_ref[...] + x2[None, :]) + xc2
    sqc = jnp.maximum(sq, 1e-12)
    dist = sqc * jax.lax.rsqrt(sqc)
    dist_ref[...] = dist[:K, :]
    assign_ref[...] = jnp.argmin(sqc, axis=0).astype(jnp.int32)


@jax.jit
def kernel(data, centroids):
    c_pad = jnp.full((K_PAD, D), _PAD_VAL, jnp.float32).at[:K].set(centroids)
    cs = -2.0 * c_pad
    dist_t, assign = pl.pallas_call(
        _kmeans_body,
        grid=(N // TN,),
        in_specs=[
            pl.BlockSpec((TN, D), lambda i: (i, 0)),
            pl.BlockSpec((K_PAD, D), lambda i: (0, 0)),
        ],
        out_specs=[
            pl.BlockSpec((K, TN), lambda i: (0, i)),
            pl.BlockSpec((TN,), lambda i: (i,)),
        ],
        out_shape=[
            jax.ShapeDtypeStruct((K, N), jnp.float32),
            jax.ShapeDtypeStruct((N,), jnp.int32),
        ],
        scratch_shapes=[pltpu.VMEM((K_PAD, 1), jnp.float32)],
    )(data, cs)
    return dist_t.T, assign


# speedup vs baseline: 3.7650x; 1.7572x over previous
import jax
import jax.numpy as jnp
from jax.experimental import pallas as pl
from jax.experimental.pallas import tpu as pltpu

N, D, K = 16384, 128, 1000
K_PAD = 1024
TN = 1024

_PAD_VAL = 1e15


def _kmeans_body(x_ref, cs_ref, dist_ref, assign_ref, c2_ref):
    @pl.when(pl.program_id(0) == 0)
    def _():
        cs0 = cs_ref[...]
        c2_ref[...] = 0.25 * jnp.sum(cs0 * cs0, axis=1, keepdims=True)

    x = x_ref[...]
    cs = cs_ref[...]
    xc2 = jax.lax.dot_general(
        cs, x, (((1,), (1,)), ((), ())),
        preferred_element_type=jnp.float32)
    x2 = jnp.sum(x * x, axis=1)
    sq = (c2_ref[...] + x2[None, :]) + xc2
    sqc = jnp.maximum(sq, 1e-12)
    dist = sqc * jax.lax.rsqrt(sqc)
    dist_ref[...] = dist[:K, :]
    assign_ref[...] = jnp.argmin(sqc, axis=0).astype(jnp.int32)


@jax.jit
def kernel(data, centroids):
    c_pad = jnp.full((K_PAD, D), _PAD_VAL, jnp.float32).at[:K].set(centroids)
    cs = -2.0 * c_pad
    dist_t, assign = pl.pallas_call(
        _kmeans_body,
        grid=(N // TN,),
        in_specs=[
            pl.BlockSpec((TN, D), lambda i: (i, 0)),
            pl.BlockSpec((K_PAD, D), lambda i: (0, 0)),
        ],
        out_specs=[
            pl.BlockSpec((K, TN), lambda i: (0, i)),
            pl.BlockSpec((TN,), lambda i: (i,)),
        ],
        out_shape=[
            jax.ShapeDtypeStruct((K, N), jnp.float32),
            jax.ShapeDtypeStruct((N,), jnp.int32),
        ],
        scratch_shapes=[pltpu.VMEM((K_PAD, 1), jnp.float32)],
    )(data, cs)
    return dist_t.T, assign


# speedup vs baseline: 3.9293x; 1.0436x over previous
import jax
import jax.numpy as jnp
from jax.experimental import pallas as pl
from jax.experimental.pallas import tpu as pltpu

N, D, K = 16384, 128, 1000
K_PAD = 1024
TN = 1024

_PAD_VAL = 1e15


def _kmeans_body(x_ref, c_ref, dist_ref, assign_ref, cs_ref, c2_ref):
    @pl.when(pl.program_id(0) == 0)
    def _():
        cs_ref[:K, :] = -2.0 * c_ref[...]
        cs_ref[K:, :] = jnp.full((K_PAD - K, D), -2.0 * _PAD_VAL, jnp.float32)
        cs0 = cs_ref[...]
        c2_ref[...] = 0.25 * jnp.sum(cs0 * cs0, axis=1, keepdims=True)

    x = x_ref[...]
    cs = cs_ref[...]
    xc2 = jax.lax.dot_general(
        cs, x, (((1,), (1,)), ((), ())),
        preferred_element_type=jnp.float32)
    x2 = jnp.sum(x * x, axis=1)
    sq = (c2_ref[...] + x2[None, :]) + xc2
    sqc = jnp.maximum(sq, 1e-12)
    dist = sqc * jax.lax.rsqrt(sqc)
    dist_ref[...] = dist[:K, :]
    assign_ref[...] = jnp.argmin(sqc, axis=0).astype(jnp.int32)


@jax.jit
def kernel(data, centroids):
    dist_t, assign = pl.pallas_call(
        _kmeans_body,
        grid=(N // TN,),
        in_specs=[
            pl.BlockSpec((TN, D), lambda i: (i, 0)),
            pl.BlockSpec((K, D), lambda i: (0, 0)),
        ],
        out_specs=[
            pl.BlockSpec((K, TN), lambda i: (0, i)),
            pl.BlockSpec((TN,), lambda i: (i,)),
        ],
        out_shape=[
            jax.ShapeDtypeStruct((K, N), jnp.float32),
            jax.ShapeDtypeStruct((N,), jnp.int32),
        ],
        scratch_shapes=[
            pltpu.VMEM((K_PAD, D), jnp.float32),
            pltpu.VMEM((K_PAD, 1), jnp.float32),
        ],
    )(data, centroids)
    return dist_t.T, assign


# speedup vs baseline: 4.2841x; 1.0903x over previous
import jax
import jax.numpy as jnp
from jax.experimental import pallas as pl
from jax.experimental.pallas import tpu as pltpu

N, D, K = 16384, 128, 1000
K_PAD = 1024
TN = 2048

_PAD_VAL = 1e15


def _kmeans_body(x_ref, c_ref, dist_ref, assign_ref, cs_ref, c2_ref):
    @pl.when(pl.program_id(0) == 0)
    def _():
        cs_ref[:K, :] = -2.0 * c_ref[...]
        cs_ref[K:, :] = jnp.full((K_PAD - K, D), -2.0 * _PAD_VAL, jnp.float32)
        cs0 = cs_ref[...]
        c2_ref[...] = 0.25 * jnp.sum(cs0 * cs0, axis=1, keepdims=True)

    x = x_ref[...]
    cs = cs_ref[...]
    xc2 = jax.lax.dot_general(
        cs, x, (((1,), (1,)), ((), ())),
        preferred_element_type=jnp.float32)
    x2 = jnp.sum(x * x, axis=1)
    sq = (c2_ref[...] + x2[None, :]) + xc2
    sqc = jnp.maximum(sq, 1e-12)
    dist = sqc * jax.lax.rsqrt(sqc)
    dist_ref[...] = dist[:K, :]
    assign_ref[...] = jnp.argmin(sqc, axis=0).astype(jnp.int32)


@jax.jit
def kernel(data, centroids):
    dist_t, assign = pl.pallas_call(
        _kmeans_body,
        grid=(N // TN,),
        in_specs=[
            pl.BlockSpec((TN, D), lambda i: (i, 0)),
            pl.BlockSpec((K, D), lambda i: (0, 0)),
        ],
        out_specs=[
            pl.BlockSpec((K, TN), lambda i: (0, i)),
            pl.BlockSpec((TN,), lambda i: (i,)),
        ],
        out_shape=[
            jax.ShapeDtypeStruct((K, N), jnp.float32),
            jax.ShapeDtypeStruct((N,), jnp.int32),
        ],
        scratch_shapes=[
            pltpu.VMEM((K_PAD, D), jnp.float32),
            pltpu.VMEM((K_PAD, 1), jnp.float32),
        ],
    )(data, centroids)
    return dist_t.T, assign
